# baseline (speedup 1.0000x reference)
import dataclasses
import functools

import jax
import jax.numpy as jnp
from jax import lax
from jax.experimental import pallas as pl
from jax.experimental.pallas import tpu as pltpu
from jax.experimental.pallas import tpu_sc as plsc

VOCAB = 30522
HIDDEN = 768
HALF = HIDDEN // 2
MAX_POS = 512
BATCH = 32
SEQ = 512
EPS = 1e-12

NC = 2
NS = 16
NW = NC * NS

NCH = 4
B_CH = BATCH // NCH
T_CH = B_CH * SEQ
B_PER_W = T_CH // NW
G = 32
NCHUNK = B_PER_W // G

LN_ROWS = 512


def _sc_gather_pack(word_emb, idx3):
    mesh = plsc.VectorSubcoreMesh(core_axis_name="c", subcore_axis_name="s")
    cp = pltpu.CompilerParams()
    if "needs_layout_passes" in pltpu.CompilerParams.__dataclass_fields__:
        cp = dataclasses.replace(cp, needs_layout_passes=False)

    @functools.partial(
        pl.kernel,
        mesh=mesh,
        compiler_params=cp,
        out_type=jax.ShapeDtypeStruct((T_CH, HALF), jnp.int32),
        scratch_types=[
            pltpu.VMEM((NCHUNK, G), jnp.int32),
            pltpu.VMEM((G, HIDDEN), jnp.float32),
            pltpu.VMEM((G, HIDDEN), jnp.float32),
            pltpu.VMEM((G, HALF), jnp.int32),
            pltpu.VMEM((G, HALF), jnp.int32),
            pltpu.SemaphoreType.DMA,
            pltpu.SemaphoreType.DMA,
            pltpu.SemaphoreType.DMA,
            pltpu.SemaphoreType.DMA,
        ],
    )
    def k(table_hbm, idx_hbm, out_hbm, idx_v, rows0, rows1, pk0, pk1,
          gsem0, gsem1, ssem0, ssem1):
        wid = lax.axis_index("s") * NC + lax.axis_index("c")
        base = wid * B_PER_W
        pltpu.sync_copy(idx_hbm.at[wid], idx_v)
        rows = (rows0, rows1)
        pks = (pk0, pk1)
        gsems = (gsem0, gsem1)
        ssems = (ssem0, ssem1)
        gathers = [None] * NCHUNK
        stores = [None] * NCHUNK
        gathers[0] = pltpu.async_copy(
            table_hbm.at[idx_v.at[0]], rows[0], gsems[0])
        for j in range(NCHUNK):
            if j + 1 < NCHUNK:
                gathers[j + 1] = pltpu.async_copy(
                    table_hbm.at[idx_v.at[j + 1]],
                    rows[(j + 1) % 2],
                    gsems[(j + 1) % 2],
                )
            gathers[j].wait()
            if j >= 2:
                stores[j - 2].wait()
            src = rows[j % 2]
            dst = pks[j % 2]

            @plsc.parallel_loop(0, G, unroll=4)
            def _(t):
                for i in range(HALF // 16):
                    sl = pl.ds(i * 16, 16)
                    a = src[t, sl]
                    b = src[t, pl.ds(HALF + i * 16, 16)]
                    w = plsc.bitcast(
                        plsc.pack(a, b, format=plsc.PackFormat.INTERLEAVED),
                        jnp.int32)
                    dst[t, sl] = w

            stores[j] = pltpu.async_copy(
                dst, out_hbm.at[pl.ds(base + j * G, G)], ssems[j % 2])
        for j in range(max(0, NCHUNK - 2), NCHUNK):
            stores[j].wait()

    return k(word_emb, idx3)


def _ln_body(g_ref, p_ref, gamma_ref, beta_ref, o_ref):
    w = g_ref[...]
    xa = lax.bitcast_convert_type(
        lax.shift_left(w, 16), jnp.float32) + p_ref[:, :HALF]
    xb = lax.bitcast_convert_type(
        w & jnp.int32(-65536), jnp.float32) + p_ref[:, HALF:]
    s = jnp.sum(xa, axis=1, keepdims=True) + jnp.sum(xb, axis=1, keepdims=True)
    q = (jnp.sum(xa * xa, axis=1, keepdims=True)
         + jnp.sum(xb * xb, axis=1, keepdims=True))
    mu = s * (1.0 / HIDDEN)
    var = q * (1.0 / HIDDEN) - mu * mu
    rstd = lax.rsqrt(var + EPS)
    ga = gamma_ref[...]
    be = beta_ref[...]
    sa = rstd * ga[:, :HALF]
    sb = rstd * ga[:, HALF:]
    o_ref[0, :, :HALF] = (xa - mu) * sa + be[:, :HALF]
    o_ref[0, :, HALF:] = (xb - mu) * sb + be[:, HALF:]


def _tc_unpack_add_ln(acc, packed, pos_emb, gamma, beta, chunk):
    rps = SEQ // LN_ROWS
    data_specs = [
        pl.BlockSpec((LN_ROWS, HALF), lambda i: (i, 0)),
        pl.BlockSpec((LN_ROWS, HIDDEN), lambda i: (i % rps, 0)),
        pl.BlockSpec((1, HIDDEN), lambda i: (0, 0)),
        pl.BlockSpec((1, HIDDEN), lambda i: (0, 0)),
    ]
    if acc is None:
        in_specs, args, aliases, body = data_specs, (), {}, _ln_body
    else:
        def body(acc_ref, *refs):
            del acc_ref
            _ln_body(*refs)

        in_specs = [pl.BlockSpec(memory_space=pl.ANY)] + data_specs
        args, aliases = (acc,), {0: 0}
    return pl.pallas_call(
        body,
        grid=(B_CH * rps,),
        in_specs=in_specs,
        out_specs=pl.BlockSpec(
            (1, LN_ROWS, HIDDEN),
            lambda i, _c=chunk: (_c * B_CH + i // rps, i % rps, 0),
        ),
        out_shape=jax.ShapeDtypeStruct((BATCH, SEQ, HIDDEN), jnp.float32),
        input_output_aliases=aliases,
    )(*args, packed, pos_emb, gamma, beta)


def kernel(input_ids, token_type_ids, word_emb, pos_emb, ln_gamma, ln_beta):
    del token_type_ids
    ids = input_ids.astype(jnp.int32).reshape(NCH, NW, NCHUNK, G)
    gamma = ln_gamma.reshape(1, HIDDEN)
    beta = ln_beta.reshape(1, HIDDEN)
    packed = [_sc_gather_pack(word_emb, ids[k]) for k in range(NCH)]
    acc = None
    for k in range(NCH):
        acc = _tc_unpack_add_ln(acc, packed[k], pos_emb, gamma, beta, k)
    return acc


# speedup vs baseline: 1.0461x; 1.0461x over previous
import dataclasses
import functools

import jax
import jax.numpy as jnp
from jax import lax
from jax.experimental import pallas as pl
from jax.experimental.pallas import tpu as pltpu
from jax.experimental.pallas import tpu_sc as plsc

VOCAB = 30522
HIDDEN = 768
HALF = HIDDEN // 2
MAX_POS = 512
BATCH = 32
SEQ = 512
EPS = 1e-12

NC = 2
NS = 16
NW = NC * NS

CHUNKS = (12, 10, 6, 4)
G = 32

LN_ROWS = 512


def _sc_gather_pack(word_emb, idx3, n_gathers):
    b_per_w = n_gathers * G
    n_tok = b_per_w * NW
    mesh = plsc.VectorSubcoreMesh(core_axis_name="c", subcore_axis_name="s")
    cp = pltpu.CompilerParams()
    if "needs_layout_passes" in pltpu.CompilerParams.__dataclass_fields__:
        cp = dataclasses.replace(cp, needs_layout_passes=False)

    @functools.partial(
        pl.kernel,
        mesh=mesh,
        compiler_params=cp,
        out_type=jax.ShapeDtypeStruct((n_tok, HALF), jnp.int32),
        scratch_types=[
            pltpu.VMEM((n_gathers, G), jnp.int32),
            pltpu.VMEM((G, HIDDEN), jnp.float32),
            pltpu.VMEM((G, HIDDEN), jnp.float32),
            pltpu.VMEM((G, HALF), jnp.int32),
            pltpu.VMEM((G, HALF), jnp.int32),
            pltpu.SemaphoreType.DMA,
            pltpu.SemaphoreType.DMA,
            pltpu.SemaphoreType.DMA,
            pltpu.SemaphoreType.DMA,
        ],
    )
    def k(table_hbm, idx_hbm, out_hbm, idx_v, rows0, rows1, pk0, pk1,
          gsem0, gsem1, ssem0, ssem1):
        NCHUNK = n_gathers
        wid = lax.axis_index("s") * NC + lax.axis_index("c")
        base = wid * b_per_w
        pltpu.sync_copy(idx_hbm.at[wid], idx_v)
        rows = (rows0, rows1)
        pks = (pk0, pk1)
        gsems = (gsem0, gsem1)
        ssems = (ssem0, ssem1)
        gathers = [None] * NCHUNK
        stores = [None] * NCHUNK
        gathers[0] = pltpu.async_copy(
            table_hbm.at[idx_v.at[0]], rows[0], gsems[0])
        for j in range(NCHUNK):
            if j + 1 < NCHUNK:
                gathers[j + 1] = pltpu.async_copy(
                    table_hbm.at[idx_v.at[j + 1]],
                    rows[(j + 1) % 2],
                    gsems[(j + 1) % 2],
                )
            gathers[j].wait()
            if j >= 2:
                stores[j - 2].wait()
            src = rows[j % 2]
            dst = pks[j % 2]

            @plsc.parallel_loop(0, G, unroll=2)
            def _(t):
                for i in range(HALF // 16):
                    sl = pl.ds(i * 16, 16)
                    a = src[t, sl]
                    b = src[t, pl.ds(HALF + i * 16, 16)]
                    w = plsc.bitcast(
                        plsc.pack(a, b, format=plsc.PackFormat.INTERLEAVED),
                        jnp.int32)
                    dst[t, sl] = w

            stores[j] = pltpu.async_copy(
                dst, out_hbm.at[pl.ds(base + j * G, G)], ssems[j % 2])
        for j in range(max(0, NCHUNK - 2), NCHUNK):
            stores[j].wait()

    return k(word_emb, idx3)


def _ln_body(g_ref, p_ref, gamma_ref, beta_ref, o_ref):
    w = g_ref[...]
    xa = lax.bitcast_convert_type(
        lax.shift_left(w, 16), jnp.float32) + p_ref[:, :HALF]
    xb = lax.bitcast_convert_type(
        w & jnp.int32(-65536), jnp.float32) + p_ref[:, HALF:]
    s = jnp.sum(xa, axis=1, keepdims=True) + jnp.sum(xb, axis=1, keepdims=True)
    q = (jnp.sum(xa * xa, axis=1, keepdims=True)
         + jnp.sum(xb * xb, axis=1, keepdims=True))
    mu = s * (1.0 / HIDDEN)
    var = q * (1.0 / HIDDEN) - mu * mu
    rstd = lax.rsqrt(var + EPS)
    ga = gamma_ref[...]
    be = beta_ref[...]
    sa = rstd * ga[:, :HALF]
    sb = rstd * ga[:, HALF:]
    o_ref[0, :, :HALF] = (xa - mu) * sa + be[:, :HALF]
    o_ref[0, :, HALF:] = (xb - mu) * sb + be[:, HALF:]


def _tc_unpack_add_ln(acc, packed, pos_emb, gamma, beta, b_off, b_ch):
    rps = SEQ // LN_ROWS
    data_specs = [
        pl.BlockSpec((LN_ROWS, HALF), lambda i: (i, 0)),
        pl.BlockSpec((LN_ROWS, HIDDEN), lambda i: (i % rps, 0)),
        pl.BlockSpec((1, HIDDEN), lambda i: (0, 0)),
        pl.BlockSpec((1, HIDDEN), lambda i: (0, 0)),
    ]
    if acc is None:
        in_specs, args, aliases, body = data_specs, (), {}, _ln_body
    else:
        def body(acc_ref, *refs):
            del acc_ref
            _ln_body(*refs)

        in_specs = [pl.BlockSpec(memory_space=pl.ANY)] + data_specs
        args, aliases = (acc,), {0: 0}
    return pl.pallas_call(
        body,
        grid=(b_ch * rps,),
        in_specs=in_specs,
        out_specs=pl.BlockSpec(
            (1, LN_ROWS, HIDDEN),
            lambda i, _b=b_off: (_b + i // rps, i % rps, 0),
        ),
        out_shape=jax.ShapeDtypeStruct((BATCH, SEQ, HIDDEN), jnp.float32),
        input_output_aliases=aliases,
    )(*args, packed, pos_emb, gamma, beta)


def kernel(input_ids, token_type_ids, word_emb, pos_emb, ln_gamma, ln_beta):
    del token_type_ids
    ids_flat = input_ids.astype(jnp.int32).reshape(-1)
    gamma = ln_gamma.reshape(1, HIDDEN)
    beta = ln_beta.reshape(1, HIDDEN)
    packed = []
    t0 = 0
    for b_ch in CHUNKS:
        n_tok = b_ch * SEQ
        n_gathers = n_tok // (NW * G)
        idx3 = ids_flat[t0:t0 + n_tok].reshape(NW, n_gathers, G)
        packed.append(_sc_gather_pack(word_emb, idx3, n_gathers))
        t0 += n_tok
    acc = None
    b_off = 0
    for k, b_ch in enumerate(CHUNKS):
        acc = _tc_unpack_add_ln(acc, packed[k], pos_emb, gamma, beta,
                                b_off, b_ch)
        b_off += b_ch
    return acc
